# baseline (speedup 1.0000x reference)
import jax
import jax.numpy as jnp
from jax.experimental import pallas as pl
from jax.experimental.pallas import tpu as pltpu

_BM = 200
_NBUF = 3


def _gcn_kernel(x_ref, w_ref, adj_hbm, bias_ref, out_ref, sup_ref, bufs, sems):
    i = pl.program_id(0)
    nsteps = pl.num_programs(0)

    def copy_in(blk, slot):
        pltpu.make_async_copy(
            adj_hbm.at[pl.ds(blk * _BM, _BM), :],
            bufs.at[slot],
            sems.at[slot],
        ).start()

    @pl.when(i == 0)
    def _():
        for j in range(_NBUF):
            copy_in(j, j)
        sup_ref[...] = jax.lax.dot_general(
            x_ref[...], w_ref[...], (((1,), (0,)), ((), ())),
            preferred_element_type=jnp.float32,
            precision=jax.lax.Precision.DEFAULT)

    slot = jax.lax.rem(i, _NBUF)
    pltpu.make_async_copy(
        adj_hbm.at[pl.ds(i * _BM, _BM), :], bufs.at[slot], sems.at[slot]
    ).wait()
    acc = jax.lax.dot_general(
        bufs[slot], sup_ref[...], (((1,), (0,)), ((), ())),
        preferred_element_type=jnp.float32,
        precision=jax.lax.Precision.DEFAULT)
    out_ref[...] = acc + bias_ref[...]

    nxt = i + _NBUF

    @pl.when(nxt < nsteps)
    def _():
        copy_in(nxt, slot)


def kernel(x, adj, weight, bias):
    n, d_in = x.shape
    d_out = weight.shape[1]
    bias2d = bias.reshape(1, d_out)

    return pl.pallas_call(
        _gcn_kernel,
        grid=(n // _BM,),
        in_specs=[
            pl.BlockSpec((n, d_in), lambda i: (0, 0)),
            pl.BlockSpec((d_in, d_out), lambda i: (0, 0)),
            pl.BlockSpec(memory_space=pl.ANY),
            pl.BlockSpec((1, d_out), lambda i: (0, 0)),
        ],
        out_specs=pl.BlockSpec((_BM, d_out), lambda i: (i, 0)),
        out_shape=jax.ShapeDtypeStruct((n, d_out), jnp.float32),
        scratch_shapes=[
            pltpu.VMEM((n, d_out), jnp.float32),
            pltpu.VMEM((_NBUF, _BM, n), jnp.float32),
            pltpu.SemaphoreType.DMA((_NBUF,)),
        ],
    )(x, weight, adj, bias2d)
